# baseline (speedup 1.0000x reference)
import jax
import jax.numpy as jnp
from jax.experimental import pallas as pl
from jax.experimental.pallas import tpu as pltpu

SPAN = 128
STRIDE = 4
LQ = 2048
HEADS = 12
DQK = 64
DV = 64
DIM = 768
KV_USED = (LQ - 1) // STRIDE + 1
SCALE = 1.0 / (DQK ** 0.5)

QT = 512
WIN = 256
NT = LQ // QT


def _dot_t(a, b):
    return jax.lax.dot_general(a, b, (((1,), (1,)), ((), ())),
                               preferred_element_type=jnp.float32)


def _band_bias(rows, cols, shift):
    r = jax.lax.broadcasted_iota(jnp.int32, (rows, cols), 0)
    c4 = STRIDE * jax.lax.broadcasted_iota(jnp.int32, (rows, cols), 1)
    d = r + shift - c4
    valid = (d >= 0) & (d < SPAN)
    return jnp.where(valid, 0.0, -jnp.inf).astype(jnp.float32)


def _softmax_av(s, vwin):
    m = jnp.max(s, axis=1, keepdims=True)
    e = jnp.exp(s - m)
    p = e / jnp.sum(e, axis=1, keepdims=True)
    return jax.lax.dot_general(p, vwin, (((1,), (0,)), ((), ())),
                               preferred_element_type=jnp.float32)


HQ = LQ // 2


def _mha_kernel(q_ref, k_ref, v_ref, wq_ref, wk_ref, wv_ref, wout_ref, out_ref,
                kf_s, vf_s):
    t = pl.program_id(0)

    @pl.when(t == 0)
    def _():
        kf_s[...] = _dot_t(k_ref[...], wk_ref[...])
        vf_s[...] = _dot_t(v_ref[...], wv_ref[...])

    Qf = _dot_t(q_ref[...], wq_ref[...])
    r = jax.lax.broadcasted_iota(jnp.int32, (HQ, KV_USED), 0) + HQ * t
    c4 = STRIDE * jax.lax.broadcasted_iota(jnp.int32, (HQ, KV_USED), 1)
    d = r - c4
    bias = jnp.where((d >= 0) & (d < SPAN), 0.0, -jnp.inf).astype(jnp.float32)
    Kf = kf_s[...]
    Vf = vf_s[...]
    ohs = []
    for h in range(HEADS):
        qh = Qf[:, h * DQK:(h + 1) * DQK]
        kh = Kf[:, h * DQK:(h + 1) * DQK]
        vh = Vf[:, h * DV:(h + 1) * DV]
        s = _dot_t(qh, kh) * SCALE + bias
        ohs.append(_softmax_av(s, vh))
    qkv = jnp.concatenate(ohs, axis=1)
    out_ref[...] = _dot_t(qkv, wout_ref[...])


def kernel(q, k, v, Wq, Wk, Wv, Wout):
    batch = q.shape[0]
    q2 = q.reshape(batch * LQ, DIM)
    k2 = k.reshape(-1, DIM)
    v2 = v.reshape(-1, DIM)
    out = pl.pallas_call(
        _mha_kernel,
        grid=(2,),
        in_specs=[
            pl.BlockSpec((LQ // 2, DIM), lambda i: (i, 0)),
            pl.BlockSpec((KV_USED, DIM), lambda i: (0, 0)),
            pl.BlockSpec((KV_USED, DIM), lambda i: (0, 0)),
            pl.BlockSpec((HEADS * DQK, DIM), lambda i: (0, 0)),
            pl.BlockSpec((HEADS * DQK, DIM), lambda i: (0, 0)),
            pl.BlockSpec((HEADS * DV, DIM), lambda i: (0, 0)),
            pl.BlockSpec((DIM, HEADS * DV), lambda i: (0, 0)),
        ],
        out_specs=pl.BlockSpec((LQ // 2, DIM), lambda i: (i, 0)),
        out_shape=jax.ShapeDtypeStruct((LQ, DIM), jnp.float32),
        scratch_shapes=[
            pltpu.VMEM((KV_USED, HEADS * DQK), jnp.float32),
            pltpu.VMEM((KV_USED, HEADS * DV), jnp.float32),
        ],
    )(q2, k2, v2, Wq, Wk, Wv, Wout)
    return out.reshape(batch, LQ, DIM)


# speedup vs baseline: 1.1542x; 1.1542x over previous
import jax
import jax.numpy as jnp
from jax.experimental import pallas as pl

SPAN = 128
STRIDE = 4
LQ = 2048
HEADS = 12
DQK = 64
DV = 64
DIM = 768
KV_USED = (LQ - 1) // STRIDE + 1
SCALE = 1.0 / (DQK ** 0.5)

QT = 512
WIN = 256
NT = LQ // QT


def _dot_t(a, b):
    return jax.lax.dot_general(a, b, (((1,), (1,)), ((), ())),
                               preferred_element_type=jnp.float32)


def _band_bias(rows, cols, shift):
    r = jax.lax.broadcasted_iota(jnp.int32, (rows, cols), 0)
    c4 = STRIDE * jax.lax.broadcasted_iota(jnp.int32, (rows, cols), 1)
    d = r + shift - c4
    valid = (d >= 0) & (d < SPAN)
    return jnp.where(valid, 0.0, -jnp.inf).astype(jnp.float32)


def _softmax_av(s, vwin):
    m = jnp.max(s, axis=1, keepdims=True)
    e = jnp.exp(s - m)
    p = e / jnp.sum(e, axis=1, keepdims=True)
    return jax.lax.dot_general(p, vwin, (((1,), (0,)), ((), ())),
                               preferred_element_type=jnp.float32)


def _mha_kernel(q_ref, k_ref, v_ref, wq_ref, wk_ref, wv_ref, wout_ref, out_ref):
    Qf = _dot_t(q_ref[...], wq_ref[...])
    Kf = _dot_t(k_ref[...], wk_ref[...])
    Vf = _dot_t(v_ref[...], wv_ref[...])
    bias = _band_bias(LQ, KV_USED, 0)
    ohs = []
    for h in range(HEADS):
        qh = Qf[:, h * DQK:(h + 1) * DQK]
        kh = Kf[:, h * DQK:(h + 1) * DQK]
        vh = Vf[:, h * DV:(h + 1) * DV]
        s = _dot_t(qh, kh) * SCALE + bias
        ohs.append(_softmax_av(s, vh))
    qkv = jnp.concatenate(ohs, axis=1)
    out_ref[...] = _dot_t(qkv, wout_ref[...])


def kernel(q, k, v, Wq, Wk, Wv, Wout):
    batch = q.shape[0]
    q2 = q.reshape(batch * LQ, DIM)
    k2 = k.reshape(-1, DIM)
    v2 = v.reshape(-1, DIM)
    out = pl.pallas_call(
        _mha_kernel,
        grid=(1,),
        in_specs=[
            pl.BlockSpec((LQ, DIM), lambda i: (0, 0)),
            pl.BlockSpec((KV_USED, DIM), lambda i: (0, 0)),
            pl.BlockSpec((KV_USED, DIM), lambda i: (0, 0)),
            pl.BlockSpec((HEADS * DQK, DIM), lambda i: (0, 0)),
            pl.BlockSpec((HEADS * DQK, DIM), lambda i: (0, 0)),
            pl.BlockSpec((HEADS * DV, DIM), lambda i: (0, 0)),
            pl.BlockSpec((DIM, HEADS * DV), lambda i: (0, 0)),
        ],
        out_specs=pl.BlockSpec((LQ, DIM), lambda i: (0, 0)),
        out_shape=jax.ShapeDtypeStruct((LQ, DIM), jnp.float32),
    )(q2, k2, v2, Wq, Wk, Wv, Wout)
    return out.reshape(batch, LQ, DIM)
